# baseline (speedup 1.0000x reference)
import functools

import jax
import jax.numpy as jnp
from jax import lax
from jax.experimental import pallas as pl
from jax.experimental.pallas import tpu as pltpu
from jax.experimental.pallas import tpu_sc as plsc

BETA_C = 0.8
LN_C = 0.1
LP_C = 1.0

NC, NS, L = 2, 16, 16
NW = NC * NS
N_TOT = 32 * 512 * 512
COLS = 512
ROWS = N_TOT // COLS

TBR = 512
TROWS = 12288
TB = TROWS // TBR

SROWS = ROWS - TROWS
R_PER = SROWS // NW
CROWS = 32
NCHUNK = R_PER // CROWS
CSLICES = COLS // L
U = 8

_mesh = plsc.VectorSubcoreMesh(core_axis_name="c", subcore_axis_name="s")


@functools.partial(
    pl.kernel,
    out_type=jax.ShapeDtypeStruct((NW * 8 * L,), jnp.float32),
    mesh=_mesh,
    scratch_types=[
        pltpu.VMEM((CROWS, COLS), jnp.float32),
        pltpu.VMEM((CROWS, COLS), jnp.float32),
        pltpu.VMEM((CROWS, COLS), jnp.int32),
        pltpu.VMEM((CROWS, COLS), jnp.int32),
        pltpu.VMEM((8 * L,), jnp.float32),
        pltpu.SemaphoreType.DMA,
        pltpu.SemaphoreType.DMA,
        pltpu.SemaphoreType.DMA,
        pltpu.SemaphoreType.DMA,
    ],
)
def _sc_sums(r_hbm, lab_hbm, out_hbm,
             rbuf0, rbuf1, labbuf0, labbuf1, accbuf,
             sr0, sr1, sl0, sl1):
    wid = lax.axis_index("s") * NC + lax.axis_index("c")
    base = TROWS + wid * R_PER
    zero = jnp.zeros((L,), jnp.float32)
    rbufs, labbufs = (rbuf0, rbuf1), (labbuf0, labbuf1)
    rsems, lsems = (sr0, sr1), (sl0, sl1)

    def issue(c, b):
        pltpu.async_copy(r_hbm.at[pl.ds(base + c * CROWS, CROWS)],
                         rbufs[b], rsems[b])
        pltpu.async_copy(lab_hbm.at[pl.ds(base + c * CROWS, CROWS)],
                         labbufs[b], lsems[b])

    issue(0, 0)
    issue(1, 1)

    def compute(b, accs):
        rbuf, labbuf = rbufs[b], labbufs[b]

        def row_body(i, accs):
            def col_body(j, accs):
                np_, sp, qp, s, q = accs
                for u in range(U):
                    off = j * (U * L) + u * L
                    r = rbuf[i, pl.ds(off, L)]
                    labf = labbuf[i, pl.ds(off, L)].astype(jnp.float32)
                    r2 = r * r
                    np_ = np_ + labf
                    sp = sp + r * labf
                    qp = qp + r2 * labf
                    s = s + r
                    q = q + r2
                return (np_, sp, qp, s, q)

            return lax.fori_loop(0, CSLICES // U, col_body, accs)

        return lax.fori_loop(0, CROWS, row_body, accs)

    def wait(b):
        pltpu.make_async_copy(r_hbm.at[pl.ds(0, CROWS)], rbufs[b],
                              rsems[b]).wait()
        pltpu.make_async_copy(lab_hbm.at[pl.ds(0, CROWS)], labbufs[b],
                              lsems[b]).wait()

    def pair_body(g, accs):
        for b in range(2):
            wait(b)

            @pl.when(2 * g + b + 2 < NCHUNK)
            def _():
                issue(2 * g + b + 2, b)

            accs = compute(b, accs)
        return accs

    np_, sp, qp, s, q = lax.fori_loop(
        0, NCHUNK // 2, pair_body, (zero, zero, zero, zero, zero))

    for idx, v in enumerate((np_, sp, qp, s, q, zero, zero, zero)):
        accbuf[pl.ds(idx * L, L)] = v
    pltpu.sync_copy(accbuf, out_hbm.at[pl.ds(wid * 8 * L, 8 * L)])


TBRW = 128
ND = 4
NBLK = TROWS // TBRW
TR = 64


def _tc_sums_body(r_hbm, lab_hbm, o_ref, *scr):
    rbufs, labbufs = scr[0:ND], scr[ND:2 * ND]
    rsems, lsems = scr[2 * ND:3 * ND], scr[3 * ND:4 * ND]
    o_ref[...] = jnp.zeros((40, COLS), jnp.float32)

    def issue(g, b):
        pltpu.make_async_copy(r_hbm.at[pl.ds(g * TBRW, TBRW)],
                              rbufs[b], rsems[b]).start()
        pltpu.make_async_copy(lab_hbm.at[pl.ds(g * TBRW, TBRW)],
                              labbufs[b], lsems[b]).start()

    for b in range(ND):
        issue(b, b)

    def fold(rbuf, labbuf):
        np_ = sp = qp = s = q = jnp.zeros((8, COLS), jnp.float32)
        for j in range(TBRW // TR):
            r = rbuf[pl.ds(j * TR, TR), :]
            labf = labbuf[pl.ds(j * TR, TR), :].astype(jnp.float32)
            r2 = r * r
            rl = r * labf
            r2l = rl * rl
            for k in range(TR // 8):
                sl = slice(k * 8, (k + 1) * 8)
                np_ = np_ + labf[sl]
                sp = sp + rl[sl]
                qp = qp + r2l[sl]
                s = s + r[sl]
                q = q + r2[sl]
        return (np_, sp, qp, s, q)

    def round_body(t, carry):
        for b in range(ND):
            g = t * ND + b
            pltpu.make_async_copy(r_hbm.at[pl.ds(0, TBRW)],
                                  rbufs[b], rsems[b]).wait()
            pltpu.make_async_copy(lab_hbm.at[pl.ds(0, TBRW)],
                                  labbufs[b], lsems[b]).wait()

            @pl.when(g + ND < NBLK)
            def _():
                issue(g + ND, b)

            parts = fold(rbufs[b], labbufs[b])
            for qn in range(5):
                sl = slice(qn * 8, (qn + 1) * 8)
                o_ref[sl, :] = o_ref[sl, :] + parts[qn]
        return carry

    lax.fori_loop(0, NBLK // ND, round_body, 0)


_tc_sums = pl.pallas_call(
    _tc_sums_body,
    in_specs=[
        pl.BlockSpec(memory_space=pl.ANY),
        pl.BlockSpec(memory_space=pl.ANY),
    ],
    out_shape=jax.ShapeDtypeStruct((40, COLS), jnp.float32),
    scratch_shapes=(
        [pltpu.VMEM((TBRW, COLS), jnp.float32) for _ in range(ND)]
        + [pltpu.VMEM((TBRW, COLS), jnp.int32) for _ in range(ND)]
        + [pltpu.SemaphoreType.DMA for _ in range(2 * ND)]
    ),
)


def _loss_body(sc_ref, tc_ref, o_ref):
    x = sc_ref[...]
    t = jnp.sum(x, axis=0, keepdims=True)
    grp = lax.broadcasted_iota(jnp.int32, (1, 8 * L), 1) // L
    y = tc_ref[...]
    qgrp = lax.broadcasted_iota(jnp.int32, (40, COLS), 0) // 8

    def gsum(qn):
        return (jnp.sum(jnp.where(grp == qn, t, 0.0))
                + jnp.sum(jnp.where(qgrp == qn, y, 0.0)))

    np_, sp, qp, s, q = gsum(0), gsum(1), gsum(2), gsum(3), gsum(4)
    nn = jnp.float32(N_TOT) - np_
    mean_pos = sp / np_
    mean_neg = (s - sp) / nn
    var_pos = (qp - sp * mean_pos) / (np_ - 1.0)
    var_neg = ((q - qp) - (s - sp) * mean_neg) / (nn - 1.0)
    loss = (jnp.maximum(BETA_C - mean_pos, 0.0)
            + LN_C * var_pos + mean_neg + LP_C * var_neg)
    o_ref[...] = jnp.full((1, 1), loss, jnp.float32)


_finalize = pl.pallas_call(
    _loss_body,
    out_shape=jax.ShapeDtypeStruct((1, 1), jnp.float32),
)


def kernel(residues, pixel_level_labels):
    r = residues.reshape(ROWS, COLS)
    lab = pixel_level_labels.reshape(ROWS, COLS)
    parts_sc = _sc_sums(r, lab)
    parts_tc = _tc_sums(r, lab)
    out = _finalize(parts_sc.reshape(NW, 8 * L), parts_tc)
    return out.reshape(1)


# speedup vs baseline: 1.1640x; 1.1640x over previous
import functools

import jax
import jax.numpy as jnp
from jax import lax
from jax.experimental import pallas as pl
from jax.experimental.pallas import tpu as pltpu
from jax.experimental.pallas import tpu_sc as plsc

BETA_C = 0.8
LN_C = 0.1
LP_C = 1.0

NC, NS, L = 2, 16, 16
NW = NC * NS
N_TOT = 32 * 512 * 512
COLS = 512
ROWS = N_TOT // COLS

TBR = 512
TROWS = 8704
TB = TROWS // TBR

SROWS = ROWS - TROWS
R_PER = SROWS // NW
CROWS = 24
NCHUNK = R_PER // CROWS
CSLICES = COLS // L
U = 8

_mesh = plsc.VectorSubcoreMesh(core_axis_name="c", subcore_axis_name="s")


@functools.partial(
    pl.kernel,
    out_type=jax.ShapeDtypeStruct((NW * 8 * L,), jnp.float32),
    mesh=_mesh,
    scratch_types=[
        pltpu.VMEM((CROWS, COLS), jnp.float32),
        pltpu.VMEM((CROWS, COLS), jnp.float32),
        pltpu.VMEM((CROWS, COLS), jnp.int32),
        pltpu.VMEM((CROWS, COLS), jnp.int32),
        pltpu.VMEM((8 * L,), jnp.float32),
        pltpu.SemaphoreType.DMA,
        pltpu.SemaphoreType.DMA,
        pltpu.SemaphoreType.DMA,
        pltpu.SemaphoreType.DMA,
    ],
)
def _sc_sums(r_hbm, lab_hbm, out_hbm,
             rbuf0, rbuf1, labbuf0, labbuf1, accbuf,
             sr0, sr1, sl0, sl1):
    wid = lax.axis_index("s") * NC + lax.axis_index("c")
    base = TROWS + wid * R_PER
    zero = jnp.zeros((L,), jnp.float32)
    rbufs, labbufs = (rbuf0, rbuf1), (labbuf0, labbuf1)
    rsems, lsems = (sr0, sr1), (sl0, sl1)

    def issue(c, b):
        pltpu.async_copy(r_hbm.at[pl.ds(base + c * CROWS, CROWS)],
                         rbufs[b], rsems[b])
        pltpu.async_copy(lab_hbm.at[pl.ds(base + c * CROWS, CROWS)],
                         labbufs[b], lsems[b])

    issue(0, 0)
    issue(1, 1)

    def compute(b, accs):
        rbuf, labbuf = rbufs[b], labbufs[b]

        def row_body(i, accs):
            def col_body(j, accs):
                np_, sp, qp, s, q = accs
                for u in range(U):
                    off = j * (U * L) + u * L
                    r = rbuf[i, pl.ds(off, L)]
                    labf = labbuf[i, pl.ds(off, L)].astype(jnp.float32)
                    r2 = r * r
                    np_ = np_ + labf
                    sp = sp + r * labf
                    qp = qp + r2 * labf
                    s = s + r
                    q = q + r2
                return (np_, sp, qp, s, q)

            return lax.fori_loop(0, CSLICES // U, col_body, accs)

        return lax.fori_loop(0, CROWS, row_body, accs)

    def wait(b):
        pltpu.make_async_copy(r_hbm.at[pl.ds(0, CROWS)], rbufs[b],
                              rsems[b]).wait()
        pltpu.make_async_copy(lab_hbm.at[pl.ds(0, CROWS)], labbufs[b],
                              lsems[b]).wait()

    def pair_body(g, accs):
        for b in range(2):
            wait(b)

            @pl.when(2 * g + b + 2 < NCHUNK)
            def _():
                issue(2 * g + b + 2, b)

            accs = compute(b, accs)
        return accs

    np_, sp, qp, s, q = lax.fori_loop(
        0, NCHUNK // 2, pair_body, (zero, zero, zero, zero, zero))

    for idx, v in enumerate((np_, sp, qp, s, q, zero, zero, zero)):
        accbuf[pl.ds(idx * L, L)] = v
    pltpu.sync_copy(accbuf, out_hbm.at[pl.ds(wid * 8 * L, 8 * L)])


def _tc_sums_body(r_ref, lab_ref, o_ref):
    i = pl.program_id(0)
    z = jnp.zeros((8, COLS), jnp.float32)
    TR = 64

    def body(j, accs):
        np_, sp, qp, s, q = accs
        r = r_ref[pl.ds(j * TR, TR), :]
        labf = lab_ref[pl.ds(j * TR, TR), :].astype(jnp.float32)
        r2 = r * r
        rl = r * labf
        r2l = rl * rl
        for k in range(TR // 8):
            sl = slice(k * 8, (k + 1) * 8)
            np_ = np_ + labf[sl]
            sp = sp + rl[sl]
            qp = qp + r2l[sl]
            s = s + r[sl]
            q = q + r2[sl]
        return (np_, sp, qp, s, q)

    np_, sp, qp, s, q = lax.fori_loop(0, TBR // TR, body, (z, z, z, z, z))
    acc = jnp.concatenate([np_, sp, qp, s, q], axis=0)

    @pl.when(i == 0)
    def _():
        o_ref[...] = acc

    @pl.when(i > 0)
    def _():
        o_ref[...] = o_ref[...] + acc


_tc_sums = pl.pallas_call(
    _tc_sums_body,
    grid=(TB,),
    in_specs=[
        pl.BlockSpec((TBR, COLS), lambda i: (i, 0)),
        pl.BlockSpec((TBR, COLS), lambda i: (i, 0)),
    ],
    out_specs=pl.BlockSpec((40, COLS), lambda i: (0, 0)),
    out_shape=jax.ShapeDtypeStruct((40, COLS), jnp.float32),
)


def _loss_body(sc_ref, tc_ref, o_ref):
    x = sc_ref[...]
    t = jnp.sum(x, axis=0, keepdims=True)
    grp = lax.broadcasted_iota(jnp.int32, (1, 8 * L), 1) // L
    y = tc_ref[...]
    qgrp = lax.broadcasted_iota(jnp.int32, (40, COLS), 0) // 8

    def gsum(qn):
        return (jnp.sum(jnp.where(grp == qn, t, 0.0))
                + jnp.sum(jnp.where(qgrp == qn, y, 0.0)))

    np_, sp, qp, s, q = gsum(0), gsum(1), gsum(2), gsum(3), gsum(4)
    nn = jnp.float32(N_TOT) - np_
    mean_pos = sp / np_
    mean_neg = (s - sp) / nn
    var_pos = (qp - sp * mean_pos) / (np_ - 1.0)
    var_neg = ((q - qp) - (s - sp) * mean_neg) / (nn - 1.0)
    loss = (jnp.maximum(BETA_C - mean_pos, 0.0)
            + LN_C * var_pos + mean_neg + LP_C * var_neg)
    o_ref[...] = jnp.full((1, 1), loss, jnp.float32)


_finalize = pl.pallas_call(
    _loss_body,
    out_shape=jax.ShapeDtypeStruct((1, 1), jnp.float32),
)


def kernel(residues, pixel_level_labels):
    r = residues.reshape(ROWS, COLS)
    lab = pixel_level_labels.reshape(ROWS, COLS)
    parts_sc = _sc_sums(r, lab)
    parts_tc = _tc_sums(r, lab)
    out = _finalize(parts_sc.reshape(NW, 8 * L), parts_tc)
    return out.reshape(1)
